# baseline (speedup 1.0000x reference)
import functools

import jax
import jax.numpy as jnp
from jax import lax
from jax.experimental import pallas as pl
from jax.experimental.pallas import tpu as pltpu
from jax.experimental.pallas import tpu_sc as plsc

_N = 16
_H = 512
_W = 512
_L = _H * _W
_M = int(0.1 * _L)

_NSC = 2
_NSUB = 16
_NW = _NSC * _NSUB
_HALF = _L * _N // _NW
_CHUNK = 16384
_NCHUNK = _HALF // _CHUNK
_BINS = 8192
_UNROLL = 8


def _dense_body(lg_ref, tg_ref, stats_ref, tn_ref, tlb_ref):
    lg = lg_ref[0]
    tg = tg_ref[0]
    m1 = jax.nn.sigmoid(lg)
    tn = (1.0 - m1) * (1.0 - tg)
    tn_ref[0] = tn
    s1 = jnp.sum(m1)
    s2 = jnp.sum(tg)
    s12 = jnp.sum(m1 * tg)
    idx = lax.broadcasted_iota(jnp.int32, (1, 4), 1)
    stats_ref[0] = jnp.where(
        idx == 0, s1, jnp.where(idx == 1, s2, jnp.where(idx == 2, s12, 0.0))
    )
    t_lb = jnp.float32(0.0)
    for k in range(1, 8):
        c_k = jnp.sum((tn >= (k / 8.0)).astype(jnp.float32))
        t_lb = jnp.where(c_k >= float(_M), jnp.float32(k / 8.0), t_lb)
    tlb_ref[0] = jnp.full((1, 16), 1.0, jnp.float32) * t_lb


def _sc_hist_body(
    tn_hbm, tlb_hbm, out_hbm, chunk_v, tlb_v, cnt_v, sum_v, cnt2_v, sum2_v
):
    wid = lax.axis_index("s") * _NSC + lax.axis_index("c")
    base = wid * _HALF
    zeros = jnp.zeros((16,), jnp.float32)
    ones = jnp.ones((16,), jnp.float32)

    def zbody(i, carry):
        cnt_v[pl.ds(i * 16, 16)] = zeros
        sum_v[pl.ds(i * 16, 16)] = zeros
        cnt2_v[pl.ds(i * 16, 16)] = zeros
        sum2_v[pl.ds(i * 16, 16)] = zeros
        return carry

    lax.fori_loop(0, _BINS // 16, zbody, 0)

    pltpu.sync_copy(tlb_hbm.at[wid // 2], tlb_v)
    blb = lax.convert_element_type(tlb_v[...] * float(_BINS), jnp.int32)

    def cbody(c, carry):
        pltpu.sync_copy(tn_hbm.at[pl.ds(base + c * _CHUNK, _CHUNK)], chunk_v)

        def ibody(i, icarry):
            for j in range(_UNROLL):
                v = chunk_v[pl.ds(i * (16 * _UNROLL) + j * 16, 16)]
                b = jnp.minimum(
                    lax.convert_element_type(v * float(_BINS), jnp.int32),
                    _BINS - 1,
                )
                msk = b >= blb
                plsc.addupdate_scatter(
                    cnt_v if j % 2 else cnt2_v, [b], ones, mask=msk
                )
                plsc.addupdate_scatter(
                    sum_v if j % 2 else sum2_v, [b], v, mask=msk
                )
            return icarry

        lax.fori_loop(0, _CHUNK // (16 * _UNROLL), ibody, 0)
        return carry

    lax.fori_loop(0, _NCHUNK, cbody, 0)
    pltpu.sync_copy(cnt_v, out_hbm.at[wid, 0])
    pltpu.sync_copy(sum_v, out_hbm.at[wid, 1])
    pltpu.sync_copy(cnt2_v, out_hbm.at[wid, 2])
    pltpu.sync_copy(sum2_v, out_hbm.at[wid, 3])


@functools.cache
def _make_sc_hist():
    mesh = plsc.VectorSubcoreMesh(
        core_axis_name="c",
        subcore_axis_name="s",
        num_cores=_NSC,
        num_subcores=_NSUB,
    )
    return pl.kernel(
        _sc_hist_body,
        out_type=jax.ShapeDtypeStruct((_NW, 4, _BINS), jnp.float32),
        mesh=mesh,
        scratch_types=[
            pltpu.VMEM((_CHUNK,), jnp.float32),
            pltpu.VMEM((16,), jnp.float32),
            pltpu.VMEM((_BINS,), jnp.float32),
            pltpu.VMEM((_BINS,), jnp.float32),
            pltpu.VMEM((_BINS,), jnp.float32),
            pltpu.VMEM((_BINS,), jnp.float32),
        ],
        compiler_params=pltpu.CompilerParams(needs_layout_passes=False),
    )


def _fin_body(hist_ref, topm_ref):
    h = hist_ref[...].reshape(_N, 2, 4, _BINS)
    cnt = h[:, 0, 0] + h[:, 1, 0] + h[:, 0, 2] + h[:, 1, 2]
    hsum = h[:, 0, 1] + h[:, 1, 1] + h[:, 0, 3] + h[:, 1, 3]
    rc = cnt
    k = 1
    while k < _BINS:
        rc = rc + jnp.concatenate(
            [rc[:, k:], jnp.zeros((_N, k), jnp.float32)], axis=1
        )
        k *= 2
    iota_b = lax.broadcasted_iota(jnp.int32, (_N, _BINS), 1)
    bstar = jnp.max(
        jnp.where(rc >= float(_M), iota_b, 0), axis=1, keepdims=True
    )
    t = lax.convert_element_type(bstar, jnp.float32) * (1.0 / float(_BINS))
    above = iota_b >= bstar
    c_above = jnp.sum(jnp.where(above, cnt, 0.0), axis=1, keepdims=True)
    s_above = jnp.sum(jnp.where(above, hsum, 0.0), axis=1, keepdims=True)
    topm_ref[...] = s_above - t * c_above + float(_M) * t


def kernel(logits, targets):
    stats, tn, tlb = pl.pallas_call(
        _dense_body,
        grid=(_N,),
        in_specs=[
            pl.BlockSpec((1, _H, _W), lambda i: (i, 0, 0)),
            pl.BlockSpec((1, _H, _W), lambda i: (i, 0, 0)),
        ],
        out_specs=[
            pl.BlockSpec((1, 1, 4), lambda i: (i, 0, 0)),
            pl.BlockSpec((1, _H, _W), lambda i: (i, 0, 0)),
            pl.BlockSpec((1, 1, 16), lambda i: (i, 0, 0)),
        ],
        out_shape=[
            jax.ShapeDtypeStruct((_N, 1, 4), jnp.float32),
            jax.ShapeDtypeStruct((_N, _H, _W), jnp.float32),
            jax.ShapeDtypeStruct((_N, 1, 16), jnp.float32),
        ],
    )(logits, targets)

    hists = _make_sc_hist()(tn.reshape(_N * _L), tlb.reshape(_N, 16))

    topm = pl.pallas_call(
        _fin_body,
        out_shape=jax.ShapeDtypeStruct((_N, 1), jnp.float32),
    )(hists)

    s1 = stats[:, 0, 0]
    s2 = stats[:, 0, 1]
    s12 = stats[:, 0, 2]
    tm = topm[:, 0]
    score = 2.0 * (s12 + 1.0) / (s1 + 2.0 * s2 - s12 + tm + 1.0)
    return (1.0 - jnp.sum(score) / _N).astype(jnp.float32)


# speedup vs baseline: 1.1082x; 1.1082x over previous
import functools

import jax
import jax.numpy as jnp
from jax import lax
from jax.experimental import pallas as pl
from jax.experimental.pallas import tpu as pltpu
from jax.experimental.pallas import tpu_sc as plsc

_N = 16
_H = 512
_W = 512
_L = _H * _W
_M = int(0.1 * _L)

_NSC = 2
_NSUB = 16
_NW = _NSC * _NSUB
_HALF = _L * _N // _NW
_CHUNK = 16384
_NCHUNK = _HALF // _CHUNK
_BINS = 8192
_UNROLL = 8


def _dense_body(lg_ref, tg_ref, stats_ref, tn_ref):
    lg = lg_ref[0]
    tg = tg_ref[0]
    m1 = jax.nn.sigmoid(lg)
    tn = (1.0 - m1) * (1.0 - tg)
    tn_ref[0] = tn
    s1 = jnp.sum(m1)
    s2 = jnp.sum(tg)
    s12 = jnp.sum(m1 * tg)
    idx = lax.broadcasted_iota(jnp.int32, (1, 4), 1)
    stats_ref[0] = jnp.where(
        idx == 0, s1, jnp.where(idx == 1, s2, jnp.where(idx == 2, s12, 0.0))
    )


def _sc_hist_body(tn_hbm, out_hbm, chunk_v, sum_v):
    wid = lax.axis_index("s") * _NSC + lax.axis_index("c")
    base = wid * _HALF
    zeros = jnp.zeros((16,), jnp.float32)

    def zbody(i, carry):
        sum_v[pl.ds(i * 16, 16)] = zeros
        return carry

    lax.fori_loop(0, _BINS // 16, zbody, 0)

    def cbody(c, carry):
        pltpu.sync_copy(tn_hbm.at[pl.ds(base + c * _CHUNK, _CHUNK)], chunk_v)

        def ibody(i, icarry):
            for j in range(_UNROLL):
                v = chunk_v[pl.ds(i * (16 * _UNROLL) + j * 16, 16)]
                b = jnp.minimum(
                    lax.convert_element_type(v * float(_BINS), jnp.int32),
                    _BINS - 1,
                )
                plsc.addupdate_scatter(sum_v, [b], v)
            return icarry

        lax.fori_loop(0, _CHUNK // (16 * _UNROLL), ibody, 0)
        return carry

    lax.fori_loop(0, _NCHUNK, cbody, 0)
    pltpu.sync_copy(sum_v, out_hbm.at[wid])


@functools.cache
def _make_sc_hist():
    mesh = plsc.VectorSubcoreMesh(
        core_axis_name="c",
        subcore_axis_name="s",
        num_cores=_NSC,
        num_subcores=_NSUB,
    )
    return pl.kernel(
        _sc_hist_body,
        out_type=jax.ShapeDtypeStruct((_NW, _BINS), jnp.float32),
        mesh=mesh,
        scratch_types=[
            pltpu.VMEM((_CHUNK,), jnp.float32),
            pltpu.VMEM((_BINS,), jnp.float32),
        ],
        compiler_params=pltpu.CompilerParams(needs_layout_passes=False),
    )


def _fin_body(hist_ref, topm_ref):
    h = hist_ref[...].reshape(_N, 2, _BINS)
    hsum = h[:, 0] + h[:, 1]
    iota_b = lax.broadcasted_iota(jnp.int32, (_N, _BINS), 1)
    cnt_lo = hsum * (
        float(_BINS) / (lax.convert_element_type(iota_b, jnp.float32) + 1.0)
    )
    rc = cnt_lo
    k = 1
    while k < _BINS:
        rc = rc + jnp.concatenate(
            [rc[:, k:], jnp.zeros((_N, k), jnp.float32)], axis=1
        )
        k *= 2
    bstar = jnp.max(
        jnp.where(rc >= float(_M), iota_b, 0), axis=1, keepdims=True
    )
    t = lax.convert_element_type(bstar, jnp.float32) * (1.0 / float(_BINS))
    above = iota_b >= bstar
    c_above = jnp.sum(jnp.where(above, cnt_lo, 0.0), axis=1, keepdims=True)
    s_above = jnp.sum(jnp.where(above, hsum, 0.0), axis=1, keepdims=True)
    topm_ref[...] = s_above - t * c_above + float(_M) * t


def kernel(logits, targets):
    stats, tn = pl.pallas_call(
        _dense_body,
        grid=(_N,),
        in_specs=[
            pl.BlockSpec((1, _H, _W), lambda i: (i, 0, 0)),
            pl.BlockSpec((1, _H, _W), lambda i: (i, 0, 0)),
        ],
        out_specs=[
            pl.BlockSpec((1, 1, 4), lambda i: (i, 0, 0)),
            pl.BlockSpec((1, _H, _W), lambda i: (i, 0, 0)),
        ],
        out_shape=[
            jax.ShapeDtypeStruct((_N, 1, 4), jnp.float32),
            jax.ShapeDtypeStruct((_N, _H, _W), jnp.float32),
        ],
    )(logits, targets)

    hists = _make_sc_hist()(tn.reshape(_N * _L))

    topm = pl.pallas_call(
        _fin_body,
        out_shape=jax.ShapeDtypeStruct((_N, 1), jnp.float32),
    )(hists)

    s1 = stats[:, 0, 0]
    s2 = stats[:, 0, 1]
    s12 = stats[:, 0, 2]
    tm = topm[:, 0]
    score = 2.0 * (s12 + 1.0) / (s1 + 2.0 * s2 - s12 + tm + 1.0)
    return (1.0 - jnp.sum(score) / _N).astype(jnp.float32)


# speedup vs baseline: 1.7867x; 1.6123x over previous
import functools

import jax
import jax.numpy as jnp
from jax import lax
from jax.experimental import pallas as pl
from jax.experimental.pallas import tpu as pltpu
from jax.experimental.pallas import tpu_sc as plsc

_N = 16
_H = 512
_W = 512
_L = _H * _W
_M = int(0.1 * _L)

_NSC = 2
_NSUB = 16
_NW = _NSC * _NSUB
_HALF = _L * _N // _NW
_CHUNK = 16384
_NCHUNK = _HALF // _CHUNK
_BINS = 8192
_UNROLL = 8


def _dense_body(lg_ref, tg_ref, stats_ref, tn_ref):
    lg = lg_ref[0]
    tg = tg_ref[0]
    m1 = jax.nn.sigmoid(lg)
    tn = (1.0 - m1) * (1.0 - tg)
    tn_ref[0] = tn
    s1 = jnp.sum(m1)
    s2 = jnp.sum(tg)
    s12 = jnp.sum(m1 * tg)
    idx = lax.broadcasted_iota(jnp.int32, (1, 4), 1)
    stats_ref[0] = jnp.where(
        idx == 0, s1, jnp.where(idx == 1, s2, jnp.where(idx == 2, s12, 0.0))
    )


def _sc_hist_body(tn_hbm, out_hbm, chunk_v, sum_v):
    wid = lax.axis_index("s") * _NSC + lax.axis_index("c")
    base = wid * _HALF
    zeros = jnp.zeros((16,), jnp.float32)

    def zbody(i, carry):
        sum_v[pl.ds(i * 16, 16)] = zeros
        return carry

    lax.fori_loop(0, _BINS // 16, zbody, 0)

    def cbody(c, carry):
        pltpu.sync_copy(tn_hbm.at[pl.ds(base + c * _CHUNK, _CHUNK)], chunk_v)

        def ibody(i, icarry):
            vs = [
                chunk_v[pl.ds(i * (16 * _UNROLL) + j * 16, 16)]
                for j in range(_UNROLL)
            ]
            bs = [
                jnp.minimum(
                    lax.convert_element_type(v * float(_BINS), jnp.int32),
                    _BINS - 1,
                )
                for v in vs
            ]
            for v, b in zip(vs, bs):
                plsc.addupdate_scatter(sum_v, [b], v)
            return icarry

        lax.fori_loop(0, _CHUNK // (16 * _UNROLL), ibody, 0)
        return carry

    lax.fori_loop(0, _NCHUNK, cbody, 0)
    pltpu.sync_copy(sum_v, out_hbm.at[wid])


@functools.cache
def _make_sc_hist():
    mesh = plsc.VectorSubcoreMesh(
        core_axis_name="c",
        subcore_axis_name="s",
        num_cores=_NSC,
        num_subcores=_NSUB,
    )
    return pl.kernel(
        _sc_hist_body,
        out_type=jax.ShapeDtypeStruct((_NW, _BINS), jnp.float32),
        mesh=mesh,
        scratch_types=[
            pltpu.VMEM((_CHUNK,), jnp.float32),
            pltpu.VMEM((_BINS,), jnp.float32),
        ],
        compiler_params=pltpu.CompilerParams(needs_layout_passes=False),
    )


def _fin_body(hist_ref, topm_ref):
    h = hist_ref[...].reshape(_N, 2, _BINS)
    hsum = h[:, 0] + h[:, 1]
    iota_b = lax.broadcasted_iota(jnp.int32, (_N, _BINS), 1)
    cnt_lo = hsum * (
        float(_BINS) / (lax.convert_element_type(iota_b, jnp.float32) + 1.0)
    )
    rc = cnt_lo
    k = 1
    while k < _BINS:
        rc = rc + jnp.concatenate(
            [rc[:, k:], jnp.zeros((_N, k), jnp.float32)], axis=1
        )
        k *= 2
    bstar = jnp.max(
        jnp.where(rc >= float(_M), iota_b, 0), axis=1, keepdims=True
    )
    t = lax.convert_element_type(bstar, jnp.float32) * (1.0 / float(_BINS))
    above = iota_b >= bstar
    c_above = jnp.sum(jnp.where(above, cnt_lo, 0.0), axis=1, keepdims=True)
    s_above = jnp.sum(jnp.where(above, hsum, 0.0), axis=1, keepdims=True)
    topm_ref[...] = s_above - t * c_above + float(_M) * t


def kernel(logits, targets):
    stats, tn = pl.pallas_call(
        _dense_body,
        grid=(_N,),
        in_specs=[
            pl.BlockSpec((1, _H, _W), lambda i: (i, 0, 0)),
            pl.BlockSpec((1, _H, _W), lambda i: (i, 0, 0)),
        ],
        out_specs=[
            pl.BlockSpec((1, 1, 4), lambda i: (i, 0, 0)),
            pl.BlockSpec((1, _H, _W), lambda i: (i, 0, 0)),
        ],
        out_shape=[
            jax.ShapeDtypeStruct((_N, 1, 4), jnp.float32),
            jax.ShapeDtypeStruct((_N, _H, _W), jnp.float32),
        ],
    )(logits, targets)

    hists = _make_sc_hist()(tn.reshape(_N * _L))

    topm = pl.pallas_call(
        _fin_body,
        out_shape=jax.ShapeDtypeStruct((_N, 1), jnp.float32),
    )(hists)

    s1 = stats[:, 0, 0]
    s2 = stats[:, 0, 1]
    s12 = stats[:, 0, 2]
    tm = topm[:, 0]
    score = 2.0 * (s12 + 1.0) / (s1 + 2.0 * s2 - s12 + tm + 1.0)
    return (1.0 - jnp.sum(score) / _N).astype(jnp.float32)


# speedup vs baseline: 2.3454x; 1.3127x over previous
import functools

import jax
import jax.numpy as jnp
from jax import lax
from jax.experimental import pallas as pl
from jax.experimental.pallas import tpu as pltpu
from jax.experimental.pallas import tpu_sc as plsc

_N = 16
_H = 512
_W = 512
_L = _H * _W
_M = int(0.1 * _L)

_NSC = 2
_NSUB = 16
_NW = _NSC * _NSUB
_HALF = _L * _N // _NW
_CHUNK = 16384
_NCHUNK = _HALF // _CHUNK
_BINS = 8192
_UNROLL = 8


def _dense_body(lg_ref, tg_ref, stats_ref, tn_ref):
    lg = lg_ref[0]
    tg = tg_ref[0]
    m1 = jax.nn.sigmoid(lg)
    tn = (1.0 - m1) * (1.0 - tg)
    tn_ref[0] = tn
    s1 = jnp.sum(m1)
    s2 = jnp.sum(tg)
    s12 = jnp.sum(m1 * tg)
    idx = lax.broadcasted_iota(jnp.int32, (1, 4), 1)
    stats_ref[0] = jnp.where(
        idx == 0, s1, jnp.where(idx == 1, s2, jnp.where(idx == 2, s12, 0.0))
    )


_ROWS = _CHUNK // _W


def _sc_hist_body(tn_hbm, out_hbm, chunk_v, sum_v):
    wid = lax.axis_index("s") * _NSC + lax.axis_index("c")
    smp = wid // 2
    row0 = (wid % 2) * (_H // 2)
    zeros = jnp.zeros((16,), jnp.float32)

    def zbody(i, carry):
        sum_v[pl.ds(i * 16, 16)] = zeros
        return carry

    lax.fori_loop(0, _BINS // 16, zbody, 0)

    def cbody(c, carry):
        pltpu.sync_copy(
            tn_hbm.at[smp, pl.ds(row0 + c * _ROWS, _ROWS)], chunk_v
        )

        def ibody(r, icarry):
            vs = [chunk_v[r, pl.ds(j * 16, 16)] for j in range(_W // 16)]
            bs = [
                jnp.minimum(
                    lax.convert_element_type(v * float(_BINS), jnp.int32),
                    _BINS - 1,
                )
                for v in vs
            ]
            for v, b in zip(vs, bs):
                plsc.addupdate_scatter(sum_v, [b], v)
            return icarry

        lax.fori_loop(0, _ROWS, ibody, 0)
        return carry

    lax.fori_loop(0, _NCHUNK, cbody, 0)
    pltpu.sync_copy(sum_v, out_hbm.at[wid])


@functools.cache
def _make_sc_hist():
    mesh = plsc.VectorSubcoreMesh(
        core_axis_name="c",
        subcore_axis_name="s",
        num_cores=_NSC,
        num_subcores=_NSUB,
    )
    return pl.kernel(
        _sc_hist_body,
        out_type=jax.ShapeDtypeStruct((_NW, _BINS), jnp.float32),
        mesh=mesh,
        scratch_types=[
            pltpu.VMEM((_ROWS, _W), jnp.float32),
            pltpu.VMEM((_BINS,), jnp.float32),
        ],
        compiler_params=pltpu.CompilerParams(needs_layout_passes=False),
    )


def _fin_body(hist_ref, topm_ref):
    h = hist_ref[...].reshape(_N, 2, _BINS)
    hsum = h[:, 0] + h[:, 1]
    iota_b = lax.broadcasted_iota(jnp.int32, (_N, _BINS), 1)
    cnt_lo = hsum * (
        float(_BINS) / (lax.convert_element_type(iota_b, jnp.float32) + 1.0)
    )
    rc = cnt_lo
    k = 1
    while k < _BINS:
        rc = rc + jnp.concatenate(
            [rc[:, k:], jnp.zeros((_N, k), jnp.float32)], axis=1
        )
        k *= 2
    bstar = jnp.max(
        jnp.where(rc >= float(_M), iota_b, 0), axis=1, keepdims=True
    )
    t = lax.convert_element_type(bstar, jnp.float32) * (1.0 / float(_BINS))
    above = iota_b >= bstar
    c_above = jnp.sum(jnp.where(above, cnt_lo, 0.0), axis=1, keepdims=True)
    s_above = jnp.sum(jnp.where(above, hsum, 0.0), axis=1, keepdims=True)
    topm_ref[...] = s_above - t * c_above + float(_M) * t


def kernel(logits, targets):
    stats, tn = pl.pallas_call(
        _dense_body,
        grid=(_N,),
        in_specs=[
            pl.BlockSpec((1, _H, _W), lambda i: (i, 0, 0)),
            pl.BlockSpec((1, _H, _W), lambda i: (i, 0, 0)),
        ],
        out_specs=[
            pl.BlockSpec((1, 1, 4), lambda i: (i, 0, 0)),
            pl.BlockSpec((1, _H, _W), lambda i: (i, 0, 0)),
        ],
        out_shape=[
            jax.ShapeDtypeStruct((_N, 1, 4), jnp.float32),
            jax.ShapeDtypeStruct((_N, _H, _W), jnp.float32),
        ],
    )(logits, targets)

    hists = _make_sc_hist()(tn)

    topm = pl.pallas_call(
        _fin_body,
        out_shape=jax.ShapeDtypeStruct((_N, 1), jnp.float32),
    )(hists)

    s1 = stats[:, 0, 0]
    s2 = stats[:, 0, 1]
    s12 = stats[:, 0, 2]
    tm = topm[:, 0]
    score = 2.0 * (s12 + 1.0) / (s1 + 2.0 * s2 - s12 + tm + 1.0)
    return (1.0 - jnp.sum(score) / _N).astype(jnp.float32)
